# baseline (speedup 1.0000x reference)
import functools

import jax
import jax.numpy as jnp
from jax import lax
from jax.experimental import pallas as pl
from jax.experimental.pallas import tpu as pltpu
from jax.experimental.pallas import tpu_sc as plsc

N = 10000
E = 320000
D = 128

NC = 2
NS = 16
NW = NC * NS
EW = E // NW
CHUNK = 40
NCHUNK = EW // CHUNK
EROWS = E // CHUNK
CW = 16
NPAD = 10240
NPS = NPAD // NS
ZR = 32
NB = 5


NG = 2


def _sc_aggregate(node_pb, src1, dst1, attr1):
    mesh = plsc.VectorSubcoreMesh(core_axis_name="c", subcore_axis_name="s")

    @functools.partial(
        pl.kernel,
        mesh=mesh,
        out_type=jax.ShapeDtypeStruct((NC * NPAD, D), jnp.float32),
        scratch_types=[
            pltpu.VMEM((NB, CHUNK), jnp.int32),
            pltpu.VMEM((NB, CHUNK), jnp.int32),
            pltpu.VMEM((NB, CHUNK), jnp.float32),
            pltpu.VMEM((NG, CHUNK, D // 2), jnp.int32),
            pltpu.VMEM((NB, CHUNK, D), jnp.float32),
            pltpu.VMEM_SHARED((NPAD, D), jnp.float32),
        ] + [pltpu.SemaphoreType.DMA] * (NG + 2 * NB),
        compiler_params=pltpu.CompilerParams(needs_layout_passes=False,
                                             use_tc_tiling_on_sc=False),
    )
    def agg_kernel(node_h, src_h, dst_h, attr_h, out_acc,
                   srcb, dstb, attrb, gbuf, sbuf, acc_s, *sems):
        gsem = sems[:NG]
        ssem = sems[NG:NG + NB]
        isem = sems[NG + NB:]
        c = lax.axis_index("c")
        s = lax.axis_index("s")
        wid = s * NC + c
        ebase = wid * EW

        def start_idx(k, j):
            off = ebase + j * CHUNK
            pltpu.async_copy(src_h.at[pl.ds(off, CHUNK)], srcb.at[k],
                             isem[k])
            pltpu.async_copy(dst_h.at[pl.ds(off, CHUNK)], dstb.at[k],
                             isem[k])
            pltpu.async_copy(attr_h.at[pl.ds(off, CHUNK)], attrb.at[k],
                             isem[k])

        def wait_idx(k):
            pltpu.make_async_copy(src_h.at[pl.ds(0, CHUNK)], srcb.at[k],
                                  isem[k]).wait()
            pltpu.make_async_copy(dst_h.at[pl.ds(0, CHUNK)], dstb.at[k],
                                  isem[k]).wait()
            pltpu.make_async_copy(attr_h.at[pl.ds(0, CHUNK)], attrb.at[k],
                                  isem[k]).wait()

        def start_gather(g, k):
            pltpu.async_copy(node_h.at[srcb.at[k]], gbuf.at[g], gsem[g])

        def wait_gather(g, k):
            pltpu.make_async_copy(node_h.at[srcb.at[k]], gbuf.at[g],
                                  gsem[g]).wait()

        def start_scatter(k):
            pltpu.async_copy(sbuf.at[k], acc_s.at[dstb.at[k]], ssem[k],
                             add=True)

        def wait_scatter(k):
            pltpu.make_async_copy(sbuf.at[k], acc_s.at[dstb.at[k]],
                                  ssem[k]).wait()

        start_idx(0, 0)
        start_idx(1, 1)
        wait_idx(0)
        start_gather(0, 0)

        zeros16 = jnp.zeros((16,), jnp.float32)

        def fill_zero(r, _):
            for f in range(D // 16):
                sbuf[0, r, pl.ds(f * 16, 16)] = zeros16
            return 0

        lax.fori_loop(0, CHUNK, fill_zero, 0)

        for k in range(NPS // CHUNK):
            base = s * NPS + k * CHUNK
            pltpu.sync_copy(sbuf.at[0], acc_s.at[pl.ds(base, CHUNK)])

        plsc.subcore_barrier()

        himask = jnp.full((16,), -65536, jnp.int32)
        sh16 = jnp.full((16,), 16, jnp.int32)

        def scale_chunk(g, k):
            def half_body(h, _):
                e0 = h * 20
                av_a = attrb[k, pl.ds(e0, 16)]
                av_b = attrb[k, pl.ds(e0 + 4, 16)]
                for jj in range(20):
                    av = av_a if jj < 16 else av_b
                    lane = jj if jj < 16 else jj - 4
                    avj = jnp.full((16,), av[lane], jnp.float32)
                    e = e0 + jj
                    for f in range(D // 32):
                        w = gbuf[g, e, pl.ds(f * 16, 16)]
                        lo = plsc.bitcast(lax.shift_left(w, sh16),
                                          jnp.float32)
                        hi = plsc.bitcast(w & himask, jnp.float32)
                        sbuf[k, e, pl.ds(f * 32, 16)] = lo * avj
                        sbuf[k, e, pl.ds(f * 32 + 16, 16)] = hi * avj
                return 0

            lax.fori_loop(0, 2, half_body, 0)

        def outer_body(i, _):
            for u in range(2 * NB):
                j = i * (2 * NB) + u
                g0 = u % NG
                g1 = (u + 1) % NG
                k5 = u % NB
                k1 = (u + 1) % NB
                k2 = (u + 2) % NB

                @pl.when(j + 2 < NCHUNK)
                def _():
                    @pl.when(j >= 3)
                    def _():
                        wait_scatter(k2)

                    start_idx(k2, j + 2)

                @pl.when(j + 1 < NCHUNK)
                def _():
                    wait_idx(k1)
                    start_gather(g1, k1)

                wait_gather(g0, k5)
                scale_chunk(g0, k5)
                start_scatter(k5)
            return 0

        lax.fori_loop(0, NCHUNK // (2 * NB), outer_body, 0)

        for k in range(NB):
            wait_scatter(k)

        plsc.subcore_barrier()

        obase = c * NPAD + s * NPS
        pltpu.sync_copy(acc_s.at[pl.ds(s * NPS, NPS)],
                        out_acc.at[pl.ds(obase, NPS)])

    return agg_kernel(node_pb, src1, dst1, attr1)


def _sc_count(dst2):
    mesh = plsc.VectorSubcoreMesh(core_axis_name="c", subcore_axis_name="s")

    @functools.partial(
        pl.kernel,
        mesh=mesh,
        out_type=jax.ShapeDtypeStruct((NC * NPAD, CW), jnp.float32),
        scratch_types=[
            pltpu.VMEM((NCHUNK, CHUNK), jnp.int32),
            pltpu.VMEM((CHUNK, CW), jnp.float32),
            pltpu.VMEM((ZR, CW), jnp.float32),
            pltpu.VMEM_SHARED((NPAD, CW), jnp.float32),
        ] + [pltpu.SemaphoreType.DMA] * NB,
        compiler_params=pltpu.CompilerParams(use_tc_tiling_on_sc=False),
    )
    def cnt_kernel(dst_h, out_cnt, dstb, ones_v, zcnt_v, cnt_s, *csem):
        c = lax.axis_index("c")
        s = lax.axis_index("s")
        wid = s * NC + c
        row0 = wid * NCHUNK

        pltpu.sync_copy(dst_h.at[pl.ds(row0, NCHUNK)], dstb)

        zeros16 = jnp.zeros((16,), jnp.float32)
        ones16 = jnp.ones((16,), jnp.float32)

        def fill_z(r, _):
            zcnt_v[r, pl.ds(0, 16)] = zeros16
            return 0

        lax.fori_loop(0, ZR, fill_z, 0)

        def fill_ones(r, _):
            ones_v[r, pl.ds(0, 16)] = ones16
            return 0

        lax.fori_loop(0, CHUNK, fill_ones, 0)

        for k in range(NPS // ZR):
            base = s * NPS + k * ZR
            pltpu.sync_copy(zcnt_v, cnt_s.at[pl.ds(base, ZR)])

        plsc.subcore_barrier()

        def wait_cnt(k):
            pltpu.make_async_copy(ones_v, cnt_s.at[dstb.at[0]],
                                  csem[k]).wait()

        def outer_body(i, _):
            for k in range(NB):
                j = i * NB + k

                @pl.when(j >= NB)
                def _():
                    wait_cnt(k)

                pltpu.async_copy(ones_v, cnt_s.at[dstb.at[j]], csem[k],
                                 add=True)
            return 0

        lax.fori_loop(0, NCHUNK // NB, outer_body, 0)

        for k in range(NB):
            wait_cnt(k)

        plsc.subcore_barrier()

        obase = c * NPAD + s * NPS
        pltpu.sync_copy(cnt_s.at[pl.ds(s * NPS, NPS)],
                        out_cnt.at[pl.ds(obase, NPS)])

    return cnt_kernel(dst2)


R = 512
NBLK = NPAD // R


def _tc_body(node_ref, a0_ref, a1_ref, c0_ref, c1_ref,
             wrel_ref, brel_ref, wroot_ref, lnw_ref, lnb_ref, out_ref):
    psum = a0_ref[...] + a1_ref[...]
    cnt = c0_ref[...][:, 0:1] + c1_ref[...][:, 0:1]
    agg = psum / jnp.clip(cnt, 1.0, None)
    dn = (((1,), (1,)), ((), ()))
    conv = (lax.dot_general(agg, wrel_ref[...], dn,
                            preferred_element_type=jnp.float32)
            + brel_ref[...][None, :]
            + lax.dot_general(node_ref[...], wroot_ref[...], dn,
                              preferred_element_type=jnp.float32))
    h = node_ref[...] + conv
    mean = jnp.mean(h, axis=-1, keepdims=True)
    var = jnp.mean((h - mean) ** 2, axis=-1, keepdims=True)
    hn = (h - mean) * lax.rsqrt(var + 1e-5) * lnw_ref[...][None, :] \
        + lnb_ref[...][None, :]
    out_ref[...] = jnp.maximum(hn, 0.0)


def _tc_finish(node, accf, cntf, W_rel, b_rel, W_root, ln_weight, ln_bias):
    return pl.pallas_call(
        _tc_body,
        grid=(NBLK,),
        in_specs=[
            pl.BlockSpec((R, D), lambda i: (i, 0)),
            pl.BlockSpec((R, D), lambda i: (i, 0)),
            pl.BlockSpec((R, D), lambda i: (i + NBLK, 0)),
            pl.BlockSpec((R, CW), lambda i: (i, 0)),
            pl.BlockSpec((R, CW), lambda i: (i + NBLK, 0)),
            pl.BlockSpec((D, D), lambda i: (0, 0)),
            pl.BlockSpec((D,), lambda i: (0,)),
            pl.BlockSpec((D, D), lambda i: (0, 0)),
            pl.BlockSpec((D,), lambda i: (0,)),
            pl.BlockSpec((D,), lambda i: (0,)),
        ],
        out_specs=pl.BlockSpec((R, D), lambda i: (i, 0)),
        out_shape=jax.ShapeDtypeStruct((N, D), jnp.float32),
    )(node, accf, accf, cntf, cntf, W_rel, b_rel, W_root, ln_weight, ln_bias)


def kernel(node, edge_index, edge_attr, batch_ptr,
           W_rel, b_rel, W_root, ln_weight, ln_bias):
    src1 = edge_index[0].astype(jnp.int32)
    dst1 = edge_index[1].astype(jnp.int32)
    node_pb = (node.astype(jnp.bfloat16)
               .reshape(N, D // 32, 2, 16)
               .transpose(0, 1, 3, 2)
               .reshape(N, D // 2, 2))
    node_pi = lax.bitcast_convert_type(node_pb, jnp.int32)
    accf = _sc_aggregate(node_pi, src1, dst1, edge_attr)
    cntf = _sc_count(dst1.reshape(EROWS, CHUNK))
    return _tc_finish(node, accf, cntf, W_rel, b_rel, W_root,
                      ln_weight, ln_bias)


# speedup vs baseline: 1.5992x; 1.5992x over previous
import functools

import jax
import jax.numpy as jnp
from jax import lax
from jax.experimental import pallas as pl
from jax.experimental.pallas import tpu as pltpu
from jax.experimental.pallas import tpu_sc as plsc

N = 10000
E = 320000
D = 128

NC = 2
NS = 16
NW = NC * NS
EW = E // NW
CHUNK = 40
NCHUNK = EW // CHUNK
EROWS = E // CHUNK
CW = 16
NPAD = 10240
NPS = NPAD // NS
ZR = 32
NB = 5


def _sc_aggregate(node, src1, dst1, attr1):
    mesh = plsc.VectorSubcoreMesh(core_axis_name="c", subcore_axis_name="s")

    @functools.partial(
        pl.kernel,
        mesh=mesh,
        out_type=jax.ShapeDtypeStruct((NC * NPAD, D), jnp.float32),
        scratch_types=[
            pltpu.VMEM((NB, CHUNK), jnp.int32),
            pltpu.VMEM((NB, CHUNK), jnp.int32),
            pltpu.VMEM((NB, CHUNK), jnp.float32),
            pltpu.VMEM((NB, CHUNK, D), jnp.float32),
            pltpu.VMEM_SHARED((NPAD, D), jnp.float32),
        ] + [pltpu.SemaphoreType.DMA] * (3 * NB),
    )
    def agg_kernel(node_h, src_h, dst_h, attr_h, out_acc,
                   srcb, dstb, attrb, ringb, acc_s, *sems):
        gsem = sems[:NB]
        ssem = sems[NB:2 * NB]
        isem = sems[2 * NB:]
        c = lax.axis_index("c")
        s = lax.axis_index("s")
        wid = s * NC + c
        ebase = wid * EW

        def start_idx(k, j):
            off = ebase + j * CHUNK
            pltpu.async_copy(src_h.at[pl.ds(off, CHUNK)], srcb.at[k],
                             isem[k])
            pltpu.async_copy(dst_h.at[pl.ds(off, CHUNK)], dstb.at[k],
                             isem[k])
            pltpu.async_copy(attr_h.at[pl.ds(off, CHUNK)], attrb.at[k],
                             isem[k])

        def wait_idx(k):
            pltpu.make_async_copy(src_h.at[pl.ds(0, CHUNK)], srcb.at[k],
                                  isem[k]).wait()
            pltpu.make_async_copy(dst_h.at[pl.ds(0, CHUNK)], dstb.at[k],
                                  isem[k]).wait()
            pltpu.make_async_copy(attr_h.at[pl.ds(0, CHUNK)], attrb.at[k],
                                  isem[k]).wait()

        def start_gather(k):
            pltpu.async_copy(node_h.at[srcb.at[k]], ringb.at[k], gsem[k])

        def wait_gather(k):
            pltpu.make_async_copy(node_h.at[srcb.at[k]], ringb.at[k],
                                  gsem[k]).wait()

        def start_scatter(k):
            pltpu.async_copy(ringb.at[k], acc_s.at[dstb.at[k]], ssem[k],
                             add=True)

        def wait_scatter(k):
            pltpu.make_async_copy(ringb.at[k], acc_s.at[dstb.at[k]],
                                  ssem[k]).wait()

        start_idx(0, 0)
        start_idx(1, 1)
        wait_idx(0)
        start_gather(0)

        zeros16 = jnp.zeros((16,), jnp.float32)

        def fill_zero(r, _):
            for f in range(D // 16):
                ringb[NB - 1, r, pl.ds(f * 16, 16)] = zeros16
            return 0

        lax.fori_loop(0, CHUNK, fill_zero, 0)

        for k in range(NPS // CHUNK):
            base = s * NPS + k * CHUNK
            pltpu.sync_copy(ringb.at[NB - 1], acc_s.at[pl.ds(base, CHUNK)])

        plsc.subcore_barrier()

        def scale_chunk(k):
            for base, lanes in ((0, range(16)), (16, range(16)),
                                (24, range(8, 16))):
                av = attrb[k, pl.ds(base, 16)]
                for jj in lanes:
                    avj = jnp.full((16,), av[jj], jnp.float32)
                    e = base + jj
                    for f in range(D // 16):
                        ringb[k, e, pl.ds(f * 16, 16)] = (
                            ringb[k, e, pl.ds(f * 16, 16)] * avj)

        def outer_body(i, _):
            for k in range(NB):
                j = i * NB + k
                k2 = (k + 2) % NB
                k1 = (k + 1) % NB

                @pl.when(j + 2 < NCHUNK)
                def _():
                    @pl.when(j >= 3)
                    def _():
                        wait_scatter(k2)

                    start_idx(k2, j + 2)

                @pl.when(j + 1 < NCHUNK)
                def _():
                    wait_idx(k1)
                    start_gather(k1)

                wait_gather(k)
                scale_chunk(k)
                start_scatter(k)
            return 0

        lax.fori_loop(0, NCHUNK // NB, outer_body, 0)

        for k in range(NB):
            wait_scatter(k)

        plsc.subcore_barrier()

        obase = c * NPAD + s * NPS
        pltpu.sync_copy(acc_s.at[pl.ds(s * NPS, NPS)],
                        out_acc.at[pl.ds(obase, NPS)])

    return agg_kernel(node, src1, dst1, attr1)


def _sc_count(dst2):
    mesh = plsc.VectorSubcoreMesh(core_axis_name="c", subcore_axis_name="s")

    @functools.partial(
        pl.kernel,
        mesh=mesh,
        out_type=jax.ShapeDtypeStruct((NC * NPAD, CW), jnp.float32),
        scratch_types=[
            pltpu.VMEM((NCHUNK, CHUNK), jnp.int32),
            pltpu.VMEM((CHUNK, CW), jnp.float32),
            pltpu.VMEM((ZR, CW), jnp.float32),
            pltpu.VMEM_SHARED((NPAD, CW), jnp.float32),
        ] + [pltpu.SemaphoreType.DMA] * NB,
        compiler_params=pltpu.CompilerParams(use_tc_tiling_on_sc=False),
    )
    def cnt_kernel(dst_h, out_cnt, dstb, ones_v, zcnt_v, cnt_s, *csem):
        c = lax.axis_index("c")
        s = lax.axis_index("s")
        wid = s * NC + c
        row0 = wid * NCHUNK

        pltpu.sync_copy(dst_h.at[pl.ds(row0, NCHUNK)], dstb)

        zeros16 = jnp.zeros((16,), jnp.float32)
        ones16 = jnp.ones((16,), jnp.float32)

        def fill_z(r, _):
            zcnt_v[r, pl.ds(0, 16)] = zeros16
            return 0

        lax.fori_loop(0, ZR, fill_z, 0)

        def fill_ones(r, _):
            ones_v[r, pl.ds(0, 16)] = ones16
            return 0

        lax.fori_loop(0, CHUNK, fill_ones, 0)

        for k in range(NPS // ZR):
            base = s * NPS + k * ZR
            pltpu.sync_copy(zcnt_v, cnt_s.at[pl.ds(base, ZR)])

        plsc.subcore_barrier()

        def wait_cnt(k):
            pltpu.make_async_copy(ones_v, cnt_s.at[dstb.at[0]],
                                  csem[k]).wait()

        def outer_body(i, _):
            for k in range(NB):
                j = i * NB + k

                @pl.when(j >= NB)
                def _():
                    wait_cnt(k)

                pltpu.async_copy(ones_v, cnt_s.at[dstb.at[j]], csem[k],
                                 add=True)
            return 0

        lax.fori_loop(0, NCHUNK // NB, outer_body, 0)

        for k in range(NB):
            wait_cnt(k)

        plsc.subcore_barrier()

        obase = c * NPAD + s * NPS
        pltpu.sync_copy(cnt_s.at[pl.ds(s * NPS, NPS)],
                        out_cnt.at[pl.ds(obase, NPS)])

    return cnt_kernel(dst2)


R = 512
NBLK = NPAD // R


def _tc_body(node_ref, a0_ref, a1_ref, c0_ref, c1_ref,
             wrel_ref, brel_ref, wroot_ref, lnw_ref, lnb_ref, out_ref):
    psum = a0_ref[...] + a1_ref[...]
    cnt = c0_ref[...][:, 0:1] + c1_ref[...][:, 0:1]
    agg = psum / jnp.clip(cnt, 1.0, None)
    dn = (((1,), (1,)), ((), ()))
    conv = (lax.dot_general(agg, wrel_ref[...], dn,
                            preferred_element_type=jnp.float32)
            + brel_ref[...][None, :]
            + lax.dot_general(node_ref[...], wroot_ref[...], dn,
                              preferred_element_type=jnp.float32))
    h = node_ref[...] + conv
    mean = jnp.mean(h, axis=-1, keepdims=True)
    var = jnp.mean((h - mean) ** 2, axis=-1, keepdims=True)
    hn = (h - mean) * lax.rsqrt(var + 1e-5) * lnw_ref[...][None, :] \
        + lnb_ref[...][None, :]
    out_ref[...] = jnp.maximum(hn, 0.0)


def _tc_finish(node, accf, cntf, W_rel, b_rel, W_root, ln_weight, ln_bias):
    return pl.pallas_call(
        _tc_body,
        grid=(NBLK,),
        in_specs=[
            pl.BlockSpec((R, D), lambda i: (i, 0)),
            pl.BlockSpec((R, D), lambda i: (i, 0)),
            pl.BlockSpec((R, D), lambda i: (i + NBLK, 0)),
            pl.BlockSpec((R, CW), lambda i: (i, 0)),
            pl.BlockSpec((R, CW), lambda i: (i + NBLK, 0)),
            pl.BlockSpec((D, D), lambda i: (0, 0)),
            pl.BlockSpec((D,), lambda i: (0,)),
            pl.BlockSpec((D, D), lambda i: (0, 0)),
            pl.BlockSpec((D,), lambda i: (0,)),
            pl.BlockSpec((D,), lambda i: (0,)),
        ],
        out_specs=pl.BlockSpec((R, D), lambda i: (i, 0)),
        out_shape=jax.ShapeDtypeStruct((N, D), jnp.float32),
    )(node, accf, accf, cntf, cntf, W_rel, b_rel, W_root, ln_weight, ln_bias)


def kernel(node, edge_index, edge_attr, batch_ptr,
           W_rel, b_rel, W_root, ln_weight, ln_bias):
    src1 = edge_index[0].astype(jnp.int32)
    dst1 = edge_index[1].astype(jnp.int32)
    accf = _sc_aggregate(node, src1, dst1, edge_attr)
    cntf = _sc_count(dst1.reshape(EROWS, CHUNK))
    return _tc_finish(node, accf, cntf, W_rel, b_rel, W_root,
                      ln_weight, ln_bias)


# speedup vs baseline: 1.6007x; 1.0010x over previous
import functools

import jax
import jax.numpy as jnp
from jax import lax
from jax.experimental import pallas as pl
from jax.experimental.pallas import tpu as pltpu
from jax.experimental.pallas import tpu_sc as plsc

N = 10000
E = 320000
D = 128

NC = 2
NS = 16
NW = NC * NS
EW = E // NW
CHUNK = 40
NCHUNK = EW // CHUNK
EROWS = E // CHUNK
CW = 16
NPAD = 10240
NPS = NPAD // NS
ZR = 32
NB = 5


def _sc_aggregate(node, src1, dst1, attr1):
    mesh = plsc.VectorSubcoreMesh(core_axis_name="c", subcore_axis_name="s")

    @functools.partial(
        pl.kernel,
        mesh=mesh,
        out_type=jax.ShapeDtypeStruct((NC * NPAD, D), jnp.float32),
        scratch_types=[
            pltpu.VMEM((NB, CHUNK), jnp.int32),
            pltpu.VMEM((NB, CHUNK), jnp.int32),
            pltpu.VMEM((NB, CHUNK), jnp.float32),
            pltpu.VMEM((NB, CHUNK, D), jnp.float32),
            pltpu.VMEM_SHARED((NPAD, D), jnp.float32),
        ] + [pltpu.SemaphoreType.DMA] * (3 * NB),
    )
    def agg_kernel(node_h, src_h, dst_h, attr_h, out_acc,
                   srcb, dstb, attrb, ringb, acc_s, *sems):
        gsem = sems[:NB]
        ssem = sems[NB:2 * NB]
        isem = sems[2 * NB:]
        c = lax.axis_index("c")
        s = lax.axis_index("s")
        wid = s * NC + c
        ebase = wid * EW

        def start_idx(k, j):
            off = ebase + j * CHUNK
            pltpu.async_copy(src_h.at[pl.ds(off, CHUNK)], srcb.at[k],
                             isem[k])
            pltpu.async_copy(dst_h.at[pl.ds(off, CHUNK)], dstb.at[k],
                             isem[k])
            pltpu.async_copy(attr_h.at[pl.ds(off, CHUNK)], attrb.at[k],
                             isem[k])

        def wait_idx(k):
            pltpu.make_async_copy(src_h.at[pl.ds(0, CHUNK)], srcb.at[k],
                                  isem[k]).wait()
            pltpu.make_async_copy(dst_h.at[pl.ds(0, CHUNK)], dstb.at[k],
                                  isem[k]).wait()
            pltpu.make_async_copy(attr_h.at[pl.ds(0, CHUNK)], attrb.at[k],
                                  isem[k]).wait()

        def start_gather(k):
            pltpu.async_copy(node_h.at[srcb.at[k]], ringb.at[k], gsem[k])

        def wait_gather(k):
            pltpu.make_async_copy(node_h.at[srcb.at[k]], ringb.at[k],
                                  gsem[k]).wait()

        def start_scatter(k):
            pltpu.async_copy(ringb.at[k], acc_s.at[dstb.at[k]], ssem[k],
                             add=True)

        def wait_scatter(k):
            pltpu.make_async_copy(ringb.at[k], acc_s.at[dstb.at[k]],
                                  ssem[k]).wait()

        start_idx(0, 0)
        start_idx(1, 1)
        wait_idx(0)
        start_gather(0)

        zeros16 = jnp.zeros((16,), jnp.float32)

        def fill_zero(r, _):
            for f in range(D // 16):
                ringb[NB - 1, r, pl.ds(f * 16, 16)] = zeros16
            return 0

        lax.fori_loop(0, CHUNK, fill_zero, 0)

        for k in range(NPS // CHUNK):
            base = s * NPS + k * CHUNK
            pltpu.async_copy(ringb.at[NB - 1], acc_s.at[pl.ds(base, CHUNK)],
                             ssem[0])
        for k in range(NPS // CHUNK):
            pltpu.make_async_copy(ringb.at[NB - 1],
                                  acc_s.at[pl.ds(s * NPS, CHUNK)],
                                  ssem[0]).wait()

        plsc.subcore_barrier()

        def scale_chunk(k):
            for base, lanes in ((0, range(16)), (16, range(16)),
                                (24, range(8, 16))):
                av = attrb[k, pl.ds(base, 16)]
                for jj in lanes:
                    avj = jnp.full((16,), av[jj], jnp.float32)
                    e = base + jj
                    for f in range(D // 16):
                        ringb[k, e, pl.ds(f * 16, 16)] = (
                            ringb[k, e, pl.ds(f * 16, 16)] * avj)

        def outer_body(i, _):
            for k in range(NB):
                j = i * NB + k
                k2 = (k + 2) % NB
                k1 = (k + 1) % NB

                @pl.when(j + 2 < NCHUNK)
                def _():
                    @pl.when(j >= 3)
                    def _():
                        wait_scatter(k2)

                    start_idx(k2, j + 2)

                @pl.when(j + 1 < NCHUNK)
                def _():
                    wait_idx(k1)
                    start_gather(k1)

                wait_gather(k)
                scale_chunk(k)
                start_scatter(k)
            return 0

        lax.fori_loop(0, NCHUNK // NB, outer_body, 0)

        for k in range(NB):
            wait_scatter(k)

        plsc.subcore_barrier()

        obase = c * NPAD + s * NPS
        pltpu.sync_copy(acc_s.at[pl.ds(s * NPS, NPS)],
                        out_acc.at[pl.ds(obase, NPS)])

    return agg_kernel(node, src1, dst1, attr1)


def _sc_count(dst2):
    mesh = plsc.VectorSubcoreMesh(core_axis_name="c", subcore_axis_name="s")

    @functools.partial(
        pl.kernel,
        mesh=mesh,
        out_type=jax.ShapeDtypeStruct((NC * NPAD, CW), jnp.float32),
        scratch_types=[
            pltpu.VMEM((NCHUNK, CHUNK), jnp.int32),
            pltpu.VMEM((CHUNK, CW), jnp.float32),
            pltpu.VMEM((ZR, CW), jnp.float32),
            pltpu.VMEM_SHARED((NPAD, CW), jnp.float32),
        ] + [pltpu.SemaphoreType.DMA] * NB,
        compiler_params=pltpu.CompilerParams(use_tc_tiling_on_sc=False),
    )
    def cnt_kernel(dst_h, out_cnt, dstb, ones_v, zcnt_v, cnt_s, *csem):
        c = lax.axis_index("c")
        s = lax.axis_index("s")
        wid = s * NC + c
        row0 = wid * NCHUNK

        pltpu.sync_copy(dst_h.at[pl.ds(row0, NCHUNK)], dstb)

        zeros16 = jnp.zeros((16,), jnp.float32)
        ones16 = jnp.ones((16,), jnp.float32)

        def fill_z(r, _):
            zcnt_v[r, pl.ds(0, 16)] = zeros16
            return 0

        lax.fori_loop(0, ZR, fill_z, 0)

        def fill_ones(r, _):
            ones_v[r, pl.ds(0, 16)] = ones16
            return 0

        lax.fori_loop(0, CHUNK, fill_ones, 0)

        for k in range(NPS // ZR):
            base = s * NPS + k * ZR
            pltpu.sync_copy(zcnt_v, cnt_s.at[pl.ds(base, ZR)])

        plsc.subcore_barrier()

        def wait_cnt(k):
            pltpu.make_async_copy(ones_v, cnt_s.at[dstb.at[0]],
                                  csem[k]).wait()

        def outer_body(i, _):
            for k in range(NB):
                j = i * NB + k

                @pl.when(j >= NB)
                def _():
                    wait_cnt(k)

                pltpu.async_copy(ones_v, cnt_s.at[dstb.at[j]], csem[k],
                                 add=True)
            return 0

        lax.fori_loop(0, NCHUNK // NB, outer_body, 0)

        for k in range(NB):
            wait_cnt(k)

        plsc.subcore_barrier()

        obase = c * NPAD + s * NPS
        pltpu.sync_copy(cnt_s.at[pl.ds(s * NPS, NPS)],
                        out_cnt.at[pl.ds(obase, NPS)])

    return cnt_kernel(dst2)


R = 512
NBLK = NPAD // R


def _tc_body(node_ref, a0_ref, a1_ref, c0_ref, c1_ref,
             wrel_ref, brel_ref, wroot_ref, lnw_ref, lnb_ref, out_ref):
    psum = a0_ref[...] + a1_ref[...]
    cnt = c0_ref[...][:, 0:1] + c1_ref[...][:, 0:1]
    agg = psum / jnp.clip(cnt, 1.0, None)
    dn = (((1,), (1,)), ((), ()))
    conv = (lax.dot_general(agg, wrel_ref[...], dn,
                            preferred_element_type=jnp.float32)
            + brel_ref[...][None, :]
            + lax.dot_general(node_ref[...], wroot_ref[...], dn,
                              preferred_element_type=jnp.float32))
    h = node_ref[...] + conv
    mean = jnp.mean(h, axis=-1, keepdims=True)
    var = jnp.mean((h - mean) ** 2, axis=-1, keepdims=True)
    hn = (h - mean) * lax.rsqrt(var + 1e-5) * lnw_ref[...][None, :] \
        + lnb_ref[...][None, :]
    out_ref[...] = jnp.maximum(hn, 0.0)


def _tc_finish(node, accf, cntf, W_rel, b_rel, W_root, ln_weight, ln_bias):
    return pl.pallas_call(
        _tc_body,
        grid=(NBLK,),
        in_specs=[
            pl.BlockSpec((R, D), lambda i: (i, 0)),
            pl.BlockSpec((R, D), lambda i: (i, 0)),
            pl.BlockSpec((R, D), lambda i: (i + NBLK, 0)),
            pl.BlockSpec((R, CW), lambda i: (i, 0)),
            pl.BlockSpec((R, CW), lambda i: (i + NBLK, 0)),
            pl.BlockSpec((D, D), lambda i: (0, 0)),
            pl.BlockSpec((D,), lambda i: (0,)),
            pl.BlockSpec((D, D), lambda i: (0, 0)),
            pl.BlockSpec((D,), lambda i: (0,)),
            pl.BlockSpec((D,), lambda i: (0,)),
        ],
        out_specs=pl.BlockSpec((R, D), lambda i: (i, 0)),
        out_shape=jax.ShapeDtypeStruct((N, D), jnp.float32),
    )(node, accf, accf, cntf, cntf, W_rel, b_rel, W_root, ln_weight, ln_bias)


def kernel(node, edge_index, edge_attr, batch_ptr,
           W_rel, b_rel, W_root, ln_weight, ln_bias):
    src1 = edge_index[0].astype(jnp.int32)
    dst1 = edge_index[1].astype(jnp.int32)
    accf = _sc_aggregate(node, src1, dst1, edge_attr)
    cntf = _sc_count(dst1.reshape(EROWS, CHUNK))
    return _tc_finish(node, accf, cntf, W_rel, b_rel, W_root,
                      ln_weight, ln_bias)
